# baseline (speedup 1.0000x reference)
import functools

import jax
import jax.numpy as jnp
from jax import lax
from jax.experimental import pallas as pl
from jax.experimental.pallas import tpu as pltpu
from jax.experimental.pallas import tpu_sc as plsc

_N = 50000
_E = 800000
_NC = 2
_NS = 16
_CH = 256
_EPAD = 802816
_NA = _EPAD // (_NC * _NS * _CH)
_NB = _EPAD // (_NS * _CH)
_NACC = 50176
_ZCH = 64
_BN = 2000

_ZIT = (_NACC // _NS) // _ZCH


def _seg_edge_sum():
    mesh = plsc.VectorSubcoreMesh(core_axis_name="c", subcore_axis_name="s")

    scratch = [
        pltpu.VMEM((_NA, _CH), jnp.int32),
        pltpu.VMEM((_NA, _CH), jnp.int32),
        pltpu.VMEM((_CH, 8), jnp.float32),
        pltpu.VMEM((_CH, 8), jnp.float32),
        pltpu.VMEM((_ZCH, 8), jnp.float32),
        pltpu.VMEM_SHARED((_NACC, 8), jnp.float32),
        pltpu.SemaphoreType.DMA,
        pltpu.SemaphoreType.DMA,
    ]

    def body(src_hbm, dst_hbm, table_hbm, zeros_hbm, out_hbm,
             src_v, dst_v, rows0, rows1, zbuf, acc, sem0, sem1):
        c = lax.axis_index("c")
        s = lax.axis_index("s")

        pltpu.sync_copy(zeros_hbm, zbuf)
        z0 = s * (_NACC // _NS)

        def zbody(i, carry):
            pltpu.sync_copy(zbuf, acc.at[pl.ds(z0 + i * _ZCH, _ZCH)])
            return carry
        lax.fori_loop(0, _ZIT, zbody, 0)
        plsc.subcore_barrier()

        w = c * _NS + s
        pltpu.sync_copy(src_hbm.at[w], src_v)
        pltpu.sync_copy(dst_hbm.at[w], dst_v)

        bufs = (rows0, rows1)
        sems = (sem0, sem1)
        last = _NA - 1

        pltpu.async_copy(table_hbm.at[src_v.at[0]], rows0, sem0)

        def ebody(t, carry):
            for b in range(2):
                j = 2 * t + b
                jn = jnp.minimum(j + 1, last)
                pltpu.async_copy(table_hbm.at[src_v.at[jn]],
                                 bufs[1 - b], sems[1 - b])
                pltpu.make_async_copy(table_hbm.at[src_v.at[j]],
                                      bufs[b], sems[b]).wait()
                pltpu.sync_copy(bufs[b], acc.at[dst_v.at[j]], add=True)
            return carry
        lax.fori_loop(0, _NA // 2, ebody, 0)
        pltpu.make_async_copy(table_hbm.at[src_v.at[last]], rows0, sem0).wait()
        if _NA % 2:
            pltpu.sync_copy(rows0, acc.at[dst_v.at[last]], add=True)
        plsc.subcore_barrier()

        def obody(i, carry):
            pltpu.sync_copy(acc.at[pl.ds(z0 + i * _ZCH, _ZCH)], zbuf)
            pltpu.sync_copy(zbuf, out_hbm.at[c, pl.ds(z0 + i * _ZCH, _ZCH)])
            return carry
        lax.fori_loop(0, _ZIT, obody, 0)

    return pl.kernel(
        body,
        out_type=jax.ShapeDtypeStruct((_NC, _NACC, 8), jnp.float32),
        mesh=mesh,
        scratch_types=scratch,
        compiler_params=pltpu.CompilerParams(use_tc_tiling_on_sc=False),
    )


def _seg_feat_sum(nq):
    mesh = plsc.VectorSubcoreMesh(core_axis_name="c", subcore_axis_name="s")

    scratch = [
        pltpu.VMEM((_NB, _CH), jnp.int32),
        pltpu.VMEM((_NB, _CH), jnp.int32),
        pltpu.VMEM((_CH, 8), jnp.float32),
        pltpu.VMEM((_CH, 8), jnp.float32),
        pltpu.VMEM((_ZCH, 8), jnp.float32),
        pltpu.VMEM((_ZCH, 8), jnp.float32),
        pltpu.VMEM((3, 16), jnp.int32),
        pltpu.VMEM_SHARED((_NACC, 8), jnp.float32),
        pltpu.SemaphoreType.DMA,
        pltpu.SemaphoreType.DMA,
    ]

    def body(src_hbm, dst_hbm, table_hbm, zeros_hbm, moff_hbm, out_hbm,
             src_v, dst_v, rows0, rows1, zbuf, zsrc, moff_v, acc,
             sem0, sem1):
        c = lax.axis_index("c")
        s = lax.axis_index("s")

        pltpu.sync_copy(zeros_hbm, zsrc)
        z0 = s * (_NACC // _NS)

        def zbody(i, carry):
            pltpu.sync_copy(zsrc, acc.at[pl.ds(z0 + i * _ZCH, _ZCH)])
            return carry
        lax.fori_loop(0, _ZIT, zbody, 0)

        pltpu.sync_copy(src_hbm.at[s], src_v)
        pltpu.sync_copy(dst_hbm.at[s], dst_v)
        pltpu.sync_copy(moff_hbm.at[c], moff_v)
        m = moff_v[0]
        o = moff_v[1]

        def tbody(j, carry):
            for k in range(_CH // 16):
                sl = (j, pl.ds(16 * k, 16))
                src_v[sl] = src_v[sl] * m + o
            return carry
        lax.fori_loop(0, _NB, tbody, 0)
        plsc.subcore_barrier()

        bufs = (rows0, rows1)
        sems = (sem0, sem1)
        last = _NB - 1
        two = moff_v[2]

        for q in range(nq):
            pltpu.async_copy(table_hbm.at[src_v.at[0]], rows0, sem0)

            def ebody(t, carry):
                for b in range(2):
                    j = 2 * t + b
                    jn = jnp.minimum(j + 1, last)
                    pltpu.async_copy(table_hbm.at[src_v.at[jn]],
                                     bufs[1 - b], sems[1 - b])
                    pltpu.make_async_copy(table_hbm.at[src_v.at[j]],
                                          bufs[b], sems[b]).wait()
                    pltpu.sync_copy(bufs[b], acc.at[dst_v.at[j]], add=True)
                return carry
            lax.fori_loop(0, _NB // 2, ebody, 0)
            pltpu.make_async_copy(table_hbm.at[src_v.at[last]],
                                  rows0, sem0).wait()
            if _NB % 2:
                pltpu.sync_copy(rows0, acc.at[dst_v.at[last]], add=True)
            plsc.subcore_barrier()

            if q + 1 < nq:
                def obody(i, carry):
                    sl = pl.ds(z0 + i * _ZCH, _ZCH)
                    pltpu.sync_copy(acc.at[sl], zbuf)
                    pltpu.sync_copy(zbuf, out_hbm.at[q, c, sl])
                    pltpu.sync_copy(zsrc, acc.at[sl])
                    return carry
                lax.fori_loop(0, _ZIT, obody, 0)

                def ubody(j, carry):
                    for k in range(_CH // 16):
                        sl = (j, pl.ds(16 * k, 16))
                        src_v[sl] = src_v[sl] + two
                    return carry
                lax.fori_loop(0, _NB, ubody, 0)
                plsc.subcore_barrier()
            else:
                def fbody(i, carry):
                    sl = pl.ds(z0 + i * _ZCH, _ZCH)
                    pltpu.sync_copy(acc.at[sl], zbuf)
                    pltpu.sync_copy(zbuf, out_hbm.at[q, c, sl])
                    return carry
                lax.fori_loop(0, _ZIT, fbody, 0)

    return pl.kernel(
        functools.partial(body),
        out_type=jax.ShapeDtypeStruct((nq, _NC, _NACC, 8), jnp.float32),
        mesh=mesh,
        scratch_types=scratch,
        compiler_params=pltpu.CompilerParams(use_tc_tiling_on_sc=False),
    )


_seg_edge = _seg_edge_sum()
_seg_feat4 = _seg_feat_sum(4)
_seg_feat2 = _seg_feat_sum(2)


def _tc1(parts, x_aug, wl, wr, b):
    def body(p_ref, x_ref, wl_ref, wr_ref, b_ref, h_ref, inv_ref):
        sums = p_ref[0] + p_ref[1]
        inv = 1.0 / jnp.maximum(sums[:, 5:6], 1.0)
        h = (sums * inv) @ wl_ref[...] + x_ref[...] @ wr_ref[...] + b_ref[...]
        h_ref[...] = jnp.maximum(h, 0.0)
        inv_ref[...] = inv

    return pl.pallas_call(
        body,
        grid=(_N // _BN,),
        in_specs=[
            pl.BlockSpec((2, _BN, 8), lambda i: (0, i, 0)),
            pl.BlockSpec((_BN, 8), lambda i: (i, 0)),
            pl.BlockSpec((8, 64), lambda i: (0, 0)),
            pl.BlockSpec((8, 64), lambda i: (0, 0)),
            pl.BlockSpec((1, 64), lambda i: (0, 0)),
        ],
        out_specs=[
            pl.BlockSpec((_BN, 64), lambda i: (i, 0)),
            pl.BlockSpec((_BN, 1), lambda i: (i, 0)),
        ],
        out_shape=[
            jax.ShapeDtypeStruct((_N, 64), jnp.float32),
            jax.ShapeDtypeStruct((_N, 1), jnp.float32),
        ],
    )(parts, x_aug, wl, wr, b)


def _tc2(agg, h, inv, wl2, wr2, b2, wl3, wr3, b3):
    def body(a_ref, h_ref, inv_ref, wl2_ref, wr2_ref, b2_ref,
             wl3_ref, wr3_ref, b3_ref, p_ref, r_ref):
        blocks = []
        for q in range(4):
            for c in range(2):
                blocks.append(a_ref[q, c])
        agg_b = jnp.concatenate(blocks, axis=1) * inv_ref[...]
        h2 = agg_b @ wl2_ref[...] + h_ref[...] @ wr2_ref[...] + b2_ref[...]
        h2 = jnp.maximum(h2, 0.0)
        p_ref[...] = h2 @ wl3_ref[...]
        r_ref[...] = h2 @ wr3_ref[...] + b3_ref[...]

    return pl.pallas_call(
        body,
        grid=(_N // _BN,),
        in_specs=[
            pl.BlockSpec((4, 2, _BN, 8), lambda i: (0, 0, i, 0)),
            pl.BlockSpec((_BN, 64), lambda i: (i, 0)),
            pl.BlockSpec((_BN, 1), lambda i: (i, 0)),
            pl.BlockSpec((64, 64), lambda i: (0, 0)),
            pl.BlockSpec((64, 64), lambda i: (0, 0)),
            pl.BlockSpec((1, 64), lambda i: (0, 0)),
            pl.BlockSpec((64, 32), lambda i: (0, 0)),
            pl.BlockSpec((64, 32), lambda i: (0, 0)),
            pl.BlockSpec((1, 32), lambda i: (0, 0)),
        ],
        out_specs=[
            pl.BlockSpec((_BN, 32), lambda i: (i, 0)),
            pl.BlockSpec((_BN, 32), lambda i: (i, 0)),
        ],
        out_shape=[
            jax.ShapeDtypeStruct((_N, 32), jnp.float32),
            jax.ShapeDtypeStruct((_N, 32), jnp.float32),
        ],
    )(agg, h, inv, wl2, wr2, b2, wl3, wr3, b3)


def _tc3(g, inv, r):
    def body(g_ref, inv_ref, r_ref, out_ref):
        agg = jnp.concatenate(
            [g_ref[0, 0], g_ref[0, 1], g_ref[1, 0], g_ref[1, 1]], axis=1)
        out_ref[...] = agg * inv_ref[...] + r_ref[...]

    return pl.pallas_call(
        body,
        grid=(_N // _BN,),
        in_specs=[
            pl.BlockSpec((2, 2, _BN, 8), lambda i: (0, 0, i, 0)),
            pl.BlockSpec((_BN, 1), lambda i: (i, 0)),
            pl.BlockSpec((_BN, 32), lambda i: (i, 0)),
        ],
        out_specs=pl.BlockSpec((_BN, 32), lambda i: (i, 0)),
        out_shape=jax.ShapeDtypeStruct((_N, 32), jnp.float32),
    )(g, inv, r)


def _moff(mul):
    def one(c):
        return jnp.stack([jnp.full((16,), mul, jnp.int32),
                          jnp.full((16,), c, jnp.int32),
                          jnp.full((16,), 2, jnp.int32)])
    return jnp.stack([one(0), one(1)])


def kernel(x, edge_index, Wl1, Wr1, b1, Wl2, Wr2, b2, Wl3, Wr3, b3):
    src = edge_index[0]
    dst = edge_index[1]
    pad = _EPAD - _E
    srcp = jnp.concatenate([src, jnp.zeros((pad,), jnp.int32)])
    dstp = jnp.concatenate([dst, jnp.full((pad,), _N, jnp.int32)])

    srcA = srcp.reshape(_NC * _NS, _NA, _CH)
    dstA = dstp.reshape(_NC * _NS, _NA, _CH)
    srcB = srcp.reshape(_NS, _NB, _CH)
    dstB = dstp.reshape(_NS, _NB, _CH)

    ones = jnp.ones((_N, 1), jnp.float32)
    x_aug = jnp.concatenate([x, ones, jnp.zeros((_N, 2), jnp.float32)], axis=1)
    z8 = jnp.zeros((_ZCH, 8), jnp.float32)
    wpad = jnp.zeros((3, 64), jnp.float32)
    wl1p = jnp.concatenate([Wl1, wpad], axis=0)
    wr1p = jnp.concatenate([Wr1, wpad], axis=0)

    parts1 = _seg_edge(srcA, dstA, x_aug, z8)
    h, inv = _tc1(parts1, x_aug, wl1p, wr1p, b1.reshape(1, 64))
    table2 = h.reshape(8 * _N, 8)
    agg2 = _seg_feat4(srcB, dstB, table2, z8, _moff(8))
    p, r = _tc2(agg2, h, inv,
                Wl2, Wr2, b2.reshape(1, 64), Wl3, Wr3, b3.reshape(1, 32))
    table3 = p.reshape(4 * _N, 8)
    agg3 = _seg_feat2(srcB, dstB, table3, z8, _moff(4))
    return _tc3(agg3, inv, r)


# speedup vs baseline: 1.4246x; 1.4246x over previous
import functools

import jax
import jax.numpy as jnp
from jax import lax
from jax.experimental import pallas as pl
from jax.experimental.pallas import tpu as pltpu
from jax.experimental.pallas import tpu_sc as plsc

_N = 50000
_E = 800000
_NC = 2
_NS = 16
_CH = 512
_EPAD = 802816
_NA = _EPAD // (_NC * _NS * _CH)
_NACC = 50176
_ZCH = 64
_BN = 2000

_ZIT = (_NACC // _NS) // _ZCH


def _seg_sum(nq, width):
    mesh = plsc.VectorSubcoreMesh(core_axis_name="c", subcore_axis_name="s")

    scratch = [
        pltpu.VMEM((_NA, _CH), jnp.int32),
        pltpu.VMEM((_NA, _CH), jnp.int32),
        pltpu.VMEM((_CH, width), jnp.float32),
        pltpu.VMEM((_CH, width), jnp.float32),
        pltpu.VMEM((_ZCH, width), jnp.float32),
        pltpu.VMEM((_ZCH, width), jnp.float32),
        pltpu.VMEM((2, 16), jnp.int32),
        pltpu.VMEM_SHARED((_NACC, width), jnp.float32),
        pltpu.SemaphoreType.DMA,
        pltpu.SemaphoreType.DMA,
    ]

    def body(src_hbm, dst_hbm, table_hbm, zeros_hbm, moff_hbm, out_hbm,
             src_v, dst_v, rows0, rows1, zbuf, zsrc, moff_v, acc,
             sem0, sem1):
        c = lax.axis_index("c")
        s = lax.axis_index("s")

        pltpu.sync_copy(zeros_hbm, zsrc)
        z0 = s * (_NACC // _NS)

        def zbody(i, carry):
            pltpu.sync_copy(zsrc, acc.at[pl.ds(z0 + i * _ZCH, _ZCH)])
            return carry
        lax.fori_loop(0, _ZIT, zbody, 0)

        w = c * _NS + s
        pltpu.sync_copy(src_hbm.at[w], src_v)
        pltpu.sync_copy(dst_hbm.at[w], dst_v)
        pltpu.sync_copy(moff_hbm, moff_v)

        if nq > 1:
            m = moff_v[0]

            def tbody(j, carry):
                for k in range(_CH // 16):
                    sl = (j, pl.ds(16 * k, 16))
                    src_v[sl] = src_v[sl] * m
                return carry
            lax.fori_loop(0, _NA, tbody, 0)
        plsc.subcore_barrier()

        bufs = (rows0, rows1)
        sems = (sem0, sem1)
        last = _NA - 1
        one = moff_v[1]

        for q in range(nq):
            pltpu.async_copy(table_hbm.at[src_v.at[0]], rows0, sem0)

            def ebody(t, carry):
                for b in range(2):
                    j = 2 * t + b
                    jn = jnp.minimum(j + 1, last)
                    pltpu.async_copy(table_hbm.at[src_v.at[jn]],
                                     bufs[1 - b], sems[1 - b])
                    pltpu.make_async_copy(table_hbm.at[src_v.at[j]],
                                          bufs[b], sems[b]).wait()
                    pltpu.sync_copy(bufs[b], acc.at[dst_v.at[j]], add=True)
                return carry
            lax.fori_loop(0, _NA // 2, ebody, 0)
            pltpu.make_async_copy(table_hbm.at[src_v.at[last]],
                                  rows0, sem0).wait()
            if _NA % 2:
                pltpu.sync_copy(rows0, acc.at[dst_v.at[last]], add=True)
            plsc.subcore_barrier()

            if q + 1 < nq:
                def obody(i, carry):
                    sl = pl.ds(z0 + i * _ZCH, _ZCH)
                    pltpu.sync_copy(acc.at[sl], zbuf)
                    pltpu.sync_copy(zbuf, out_hbm.at[q, c, sl])
                    pltpu.sync_copy(zsrc, acc.at[sl])
                    return carry
                lax.fori_loop(0, _ZIT, obody, 0)

                def ubody(j, carry):
                    for k in range(_CH // 16):
                        sl = (j, pl.ds(16 * k, 16))
                        src_v[sl] = src_v[sl] + one
                    return carry
                lax.fori_loop(0, _NA, ubody, 0)
                plsc.subcore_barrier()
            else:
                def fbody(i, carry):
                    sl = pl.ds(z0 + i * _ZCH, _ZCH)
                    pltpu.sync_copy(acc.at[sl], zbuf)
                    pltpu.sync_copy(zbuf, out_hbm.at[q, c, sl])
                    return carry
                lax.fori_loop(0, _ZIT, fbody, 0)

    return pl.kernel(
        body,
        out_type=jax.ShapeDtypeStruct((nq, _NC, _NACC, width), jnp.float32),
        mesh=mesh,
        scratch_types=scratch,
        compiler_params=pltpu.CompilerParams(use_tc_tiling_on_sc=False),
    )


_seg_edge = _seg_sum(1, 8)
_seg_feat4 = _seg_sum(4, 16)
_seg_feat2 = _seg_sum(2, 16)


def _tc1(parts, x_aug, wl, wr, b):
    def body(p_ref, x_ref, wl_ref, wr_ref, b_ref, h_ref, inv_ref):
        sums = p_ref[0, 0] + p_ref[0, 1]
        inv = 1.0 / jnp.maximum(sums[:, 5:6], 1.0)
        h = (sums * inv) @ wl_ref[...] + x_ref[...] @ wr_ref[...] + b_ref[...]
        h_ref[...] = jnp.maximum(h, 0.0)
        inv_ref[...] = inv

    return pl.pallas_call(
        body,
        grid=(_N // _BN,),
        in_specs=[
            pl.BlockSpec((1, 2, _BN, 8), lambda i: (0, 0, i, 0)),
            pl.BlockSpec((_BN, 8), lambda i: (i, 0)),
            pl.BlockSpec((8, 64), lambda i: (0, 0)),
            pl.BlockSpec((8, 64), lambda i: (0, 0)),
            pl.BlockSpec((1, 64), lambda i: (0, 0)),
        ],
        out_specs=[
            pl.BlockSpec((_BN, 64), lambda i: (i, 0)),
            pl.BlockSpec((_BN, 1), lambda i: (i, 0)),
        ],
        out_shape=[
            jax.ShapeDtypeStruct((_N, 64), jnp.float32),
            jax.ShapeDtypeStruct((_N, 1), jnp.float32),
        ],
    )(parts, x_aug, wl, wr, b)


def _tc2(agg, h, inv, wl2, wr2, b2, wl3, wr3, b3):
    def body(a_ref, h_ref, inv_ref, wl2_ref, wr2_ref, b2_ref,
             wl3_ref, wr3_ref, b3_ref, p_ref, r_ref):
        blocks = [a_ref[q, 0] + a_ref[q, 1] for q in range(4)]
        agg_b = jnp.concatenate(blocks, axis=1) * inv_ref[...]
        h2 = agg_b @ wl2_ref[...] + h_ref[...] @ wr2_ref[...] + b2_ref[...]
        h2 = jnp.maximum(h2, 0.0)
        p_ref[...] = h2 @ wl3_ref[...]
        r_ref[...] = h2 @ wr3_ref[...] + b3_ref[...]

    return pl.pallas_call(
        body,
        grid=(_N // _BN,),
        in_specs=[
            pl.BlockSpec((4, 2, _BN, 16), lambda i: (0, 0, i, 0)),
            pl.BlockSpec((_BN, 64), lambda i: (i, 0)),
            pl.BlockSpec((_BN, 1), lambda i: (i, 0)),
            pl.BlockSpec((64, 64), lambda i: (0, 0)),
            pl.BlockSpec((64, 64), lambda i: (0, 0)),
            pl.BlockSpec((1, 64), lambda i: (0, 0)),
            pl.BlockSpec((64, 32), lambda i: (0, 0)),
            pl.BlockSpec((64, 32), lambda i: (0, 0)),
            pl.BlockSpec((1, 32), lambda i: (0, 0)),
        ],
        out_specs=[
            pl.BlockSpec((_BN, 32), lambda i: (i, 0)),
            pl.BlockSpec((_BN, 32), lambda i: (i, 0)),
        ],
        out_shape=[
            jax.ShapeDtypeStruct((_N, 32), jnp.float32),
            jax.ShapeDtypeStruct((_N, 32), jnp.float32),
        ],
    )(agg, h, inv, wl2, wr2, b2, wl3, wr3, b3)


def _tc3(g, inv, r):
    def body(g_ref, inv_ref, r_ref, out_ref):
        agg = jnp.concatenate(
            [g_ref[0, 0] + g_ref[0, 1], g_ref[1, 0] + g_ref[1, 1]], axis=1)
        out_ref[...] = agg * inv_ref[...] + r_ref[...]

    return pl.pallas_call(
        body,
        grid=(_N // _BN,),
        in_specs=[
            pl.BlockSpec((2, 2, _BN, 16), lambda i: (0, 0, i, 0)),
            pl.BlockSpec((_BN, 1), lambda i: (i, 0)),
            pl.BlockSpec((_BN, 32), lambda i: (i, 0)),
        ],
        out_specs=pl.BlockSpec((_BN, 32), lambda i: (i, 0)),
        out_shape=jax.ShapeDtypeStruct((_N, 32), jnp.float32),
    )(g, inv, r)


def _moff(mul):
    return jnp.stack([jnp.full((16,), mul, jnp.int32),
                      jnp.full((16,), 1, jnp.int32)])


def kernel(x, edge_index, Wl1, Wr1, b1, Wl2, Wr2, b2, Wl3, Wr3, b3):
    src = edge_index[0]
    dst = edge_index[1]
    pad = _EPAD - _E
    srcp = jnp.concatenate([src, jnp.zeros((pad,), jnp.int32)])
    dstp = jnp.concatenate([dst, jnp.full((pad,), _N, jnp.int32)])

    srcA = srcp.reshape(_NC * _NS, _NA, _CH)
    dstA = dstp.reshape(_NC * _NS, _NA, _CH)

    ones = jnp.ones((_N, 1), jnp.float32)
    x_aug = jnp.concatenate([x, ones, jnp.zeros((_N, 2), jnp.float32)], axis=1)
    z8 = jnp.zeros((_ZCH, 8), jnp.float32)
    z16 = jnp.zeros((_ZCH, 16), jnp.float32)
    m1 = _moff(1)
    wpad = jnp.zeros((3, 64), jnp.float32)
    wl1p = jnp.concatenate([Wl1, wpad], axis=0)
    wr1p = jnp.concatenate([Wr1, wpad], axis=0)

    parts1 = _seg_edge(srcA, dstA, x_aug, z8, m1)
    h, inv = _tc1(parts1, x_aug, wl1p, wr1p, b1.reshape(1, 64))
    table2 = h.reshape(4 * _N, 16)
    agg2 = _seg_feat4(srcA, dstA, table2, z16, _moff(4))
    p, r = _tc2(agg2, h, inv,
                Wl2, Wr2, b2.reshape(1, 64), Wl3, Wr3, b3.reshape(1, 32))
    table3 = p.reshape(2 * _N, 16)
    agg3 = _seg_feat2(srcA, dstA, table3, z16, _moff(2))
    return _tc3(agg3, inv, r)
